# Initial kernel scaffold; baseline (speedup 1.0000x reference)
import jax
import jax.numpy as jnp
from jax.experimental import pallas as pl


def kernel(feature, typeEmbed, tableEmbed, columnEmbed):
    raise NotImplementedError("write your pallas kernel here")


# speedup vs baseline: 1.9504x; 1.9504x over previous
import functools

import jax
import jax.numpy as jnp
from jax import lax
from jax.experimental import pallas as pl
from jax.experimental.pallas import tpu as pltpu
from jax.experimental.pallas import tpu_sc as plsc


def _build_sc_kernel(T, E, n_slots, slot_base):
    info = plsc.get_sparse_core_info()
    NC, NS, L = info.num_cores, info.num_subcores, info.num_lanes
    NW = NC * NS
    TPW = T // NW
    B = 128
    n_chunks = TPW // B
    D_out = n_slots * E + 2

    mesh = plsc.VectorSubcoreMesh(core_axis_name="c", subcore_axis_name="s")

    @functools.partial(
        pl.kernel,
        mesh=mesh,
        out_type=jax.ShapeDtypeStruct((T, D_out), jnp.float32),
        scratch_types=[
            pltpu.VMEM((8, B), jnp.float32),
            pltpu.VMEM((8, B), jnp.float32),
            pltpu.VMEM((B,), jnp.int32),
            pltpu.VMEM((B, E), jnp.float32),
            pltpu.VMEM((B, 2), jnp.float32),
            pltpu.SemaphoreType.DMA,
        ],
    )
    def k(featT_hbm, comb_hbm, cost_hbm, out_hbm, fA, fB, idx_v, rows_v,
          cost_v, sem):
        wid = lax.axis_index("s") * NC + lax.axis_index("c")
        t0 = wid * TPW

        def chunk_body(c, carry):
            tok0 = t0 + c * B
            pltpu.sync_copy(featT_hbm.at[pl.ds(0, 8), pl.ds(tok0, B)], fA)
            pltpu.sync_copy(featT_hbm.at[pl.ds(8, 8), pl.ds(tok0, B)], fB)
            for j in range(n_slots):
                src = fA if j < 8 else fB
                row = j % 8
                for s in range(B // L):
                    vals = src[row, pl.ds(s * L, L)]
                    idx_v[pl.ds(s * L, L)] = (
                        vals.astype(jnp.int32) + slot_base[j]
                    )
                pltpu.async_copy(comb_hbm.at[idx_v], rows_v, sem).wait()
                pltpu.sync_copy(
                    rows_v, out_hbm.at[pl.ds(tok0, B), pl.ds(j * E, E)]
                )
            pltpu.sync_copy(cost_hbm.at[pl.ds(tok0, B)], cost_v)
            pltpu.sync_copy(
                cost_v, out_hbm.at[pl.ds(tok0, B), pl.ds(n_slots * E, 2)]
            )
            return carry

        lax.fori_loop(0, n_chunks, chunk_body, 0)

    return k


def kernel(feature, typeEmbed, tableEmbed, columnEmbed):
    bt, sq, F = feature.shape
    E = typeEmbed.shape[1]
    T = bt * sq

    typeE = typeEmbed.at[0].set(0.0)
    tableE = tableEmbed.at[0].set(0.0)
    colE = columnEmbed.at[0].set(0.0)
    comb = jnp.concatenate([typeE, tableE, colE], axis=0)

    tb = typeEmbed.shape[0]
    cb = tb + tableEmbed.shape[0]
    slot_base = (0, tb, cb, cb, tb, tb, tb, cb, cb, cb)

    feat2 = feature.reshape(T, F)
    featT = jnp.pad(feat2.T, ((0, 16 - F), (0, 0)))
    cost2 = feat2[:, len(slot_base):len(slot_base) + 2]

    k = _build_sc_kernel(T, E, len(slot_base), slot_base)
    out = k(featT, comb, cost2)
    return out.reshape(bt, sq, len(slot_base) * E + 2)
